# baseline (speedup 1.0000x reference)
import functools

import jax
import jax.numpy as jnp
from jax import lax
from jax.experimental import pallas as pl
from jax.experimental.pallas import tpu as pltpu
from jax.experimental.pallas import tpu_sc as plsc

N_NODES = 10000
DIM = 128
N_EDGES = 320000

K = 128
NC, NS = 2, 16
NW = NC * NS
NB = 80
NBLK = NB * NW
E_PAD = NBLK * K
ROWS_PAD = 10112
ZB = ROWS_PAD // NS
MMB = 1000


def _mm_kernel(x_ref, wt_ref, b_ref, o_ref):
    o_ref[...] = (
        jnp.dot(x_ref[...], wt_ref[...], preferred_element_type=jnp.float32)
        + b_ref[...]
    )


def _msg_matmul(x, wt, b):
    return pl.pallas_call(
        _mm_kernel,
        grid=(N_NODES // MMB,),
        in_specs=[
            pl.BlockSpec((MMB, DIM), lambda i: (i, 0)),
            pl.BlockSpec((DIM, DIM), lambda i: (0, 0)),
            pl.BlockSpec((1, DIM), lambda i: (0, 0)),
        ],
        out_specs=pl.BlockSpec((MMB, DIM), lambda i: (i, 0)),
        out_shape=jax.ShapeDtypeStruct((N_NODES, DIM), jnp.float32),
    )(x, wt, b.reshape(1, DIM))


def _final_kernel(s_ref, x_ref, p0_ref, p1_ref, wt_ref, b_ref, o_ref):
    t = x_ref[...] * s_ref[0, 0] + p0_ref[...] + p1_ref[...]
    y = (
        jnp.dot(t, wt_ref[...], preferred_element_type=jnp.float32)
        + b_ref[...]
    )
    o_ref[...] = jnp.maximum(y, 0.0)


def _final(s, x, p0, p1, wt, b):
    return pl.pallas_call(
        _final_kernel,
        grid=(N_NODES // MMB,),
        in_specs=[
            pl.BlockSpec(memory_space=pltpu.SMEM),
            pl.BlockSpec((MMB, DIM), lambda i: (i, 0)),
            pl.BlockSpec((MMB, DIM), lambda i: (i, 0)),
            pl.BlockSpec((MMB, DIM), lambda i: (i, 0)),
            pl.BlockSpec((DIM, DIM), lambda i: (0, 0)),
            pl.BlockSpec((1, DIM), lambda i: (0, 0)),
        ],
        out_specs=pl.BlockSpec((MMB, DIM), lambda i: (i, 0)),
        out_shape=jax.ShapeDtypeStruct((N_NODES, DIM), jnp.float32),
    )(s, x, p0, p1, wt, b.reshape(1, DIM))


def _sc_scatter(msg, pk, zeros):
    mesh = plsc.VectorSubcoreMesh(core_axis_name="c", subcore_axis_name="s")

    NSPLIT = 4
    NBUF = 2 * NSPLIT
    GA = 4
    H = K // NSPLIT
    UPR = NSPLIT

    @functools.partial(
        pl.kernel,
        out_type=(jax.ShapeDtypeStruct((ROWS_PAD, DIM), jnp.float32),
                  jax.ShapeDtypeStruct((ROWS_PAD, DIM), jnp.float32)),
        mesh=mesh,
        scratch_types=(
            [pltpu.VMEM_SHARED((ROWS_PAD, DIM), jnp.float32),
             pltpu.VMEM((NB, K), jnp.int32)]
            + [pltpu.VMEM((H,), jnp.int32) for _ in range(2 * NBUF)]
            + [pltpu.VMEM((H, DIM), jnp.float32) for _ in range(NBUF)]
            + [pltpu.SemaphoreType.DMA for _ in range(2 * NBUF + 1)]
        ),
    )
    def k(msg_hbm, pk_hbm, z_hbm, out0_hbm, out1_hbm, acc, pk_all, *rest):
        ds = rest[0:NBUF]
        ss = rest[NBUF:2 * NBUF]
        rr = rest[2 * NBUF:3 * NBUF]
        gg = rest[3 * NBUF:4 * NBUF]
        sc = rest[4 * NBUF:5 * NBUF]
        zs = rest[5 * NBUF]
        cid = lax.axis_index("c")
        sid = lax.axis_index("s")

        def unpack(row, q, dbuf, sbuf):
            for c in range(0, H, 16):
                v = pk_all[row, pl.ds(q * H + c, 16)]
                dbuf[pl.ds(c, 16)] = v & 0xFFFF
                sbuf[pl.ds(c, 16)] = v >> 16

        wid = cid * NS + sid

        pltpu.async_copy(z_hbm.at[pl.ds(sid * ZB, ZB)],
                         acc.at[pl.ds(sid * ZB, ZB)], zs)
        pltpu.sync_copy(pk_hbm.at[wid], pk_all)
        for b in range(GA):
            unpack(b // UPR, b % UPR, ds[b], ss[b])
            pltpu.async_copy(msg_hbm.at[ds[b]], rr[b], gg[b])
        pltpu.make_async_copy(z_hbm.at[pl.ds(sid * ZB, ZB)],
                              acc.at[pl.ds(sid * ZB, ZB)], zs).wait()
        plsc.subcore_barrier()

        @pl.loop(0, NB, step=NBUF // UPR)
        def _(j):
            for t in range(NBUF):
                pltpu.make_async_copy(msg_hbm.at[ds[t]], rr[t], gg[t]).wait()
                pltpu.async_copy(rr[t], acc.at[ss[t]], sc[t], add=True)

                ba = (t + GA) % NBUF
                ra = (t + GA) // UPR
                qa = (t + GA) % UPR

                @pl.when(j + ra < NB)
                def _(t=t, ba=ba, ra=ra, qa=qa, j=j):
                    def _wait_sc():
                        pltpu.make_async_copy(rr[ba], acc.at[ss[ba]],
                                              sc[ba]).wait()

                    if t < GA:
                        @pl.when(j > 0)
                        def _():
                            _wait_sc()
                    else:
                        _wait_sc()
                    unpack(j + ra, qa, ds[ba], ss[ba])
                    pltpu.async_copy(msg_hbm.at[ds[ba]], rr[ba], gg[ba])

        for b in range(NBUF):
            pltpu.make_async_copy(rr[b], acc.at[ss[b]], sc[b]).wait()
        plsc.subcore_barrier()

        @pl.when(cid == 0)
        def _():
            pltpu.sync_copy(acc.at[pl.ds(sid * ZB, ZB)],
                            out0_hbm.at[pl.ds(sid * ZB, ZB)])

        @pl.when(cid == 1)
        def _():
            pltpu.sync_copy(acc.at[pl.ds(sid * ZB, ZB)],
                            out1_hbm.at[pl.ds(sid * ZB, ZB)])

    return k(msg, pk, zeros)


def kernel(X, edge_index, eps, W_msg, b_msg, W_lin, b_lin):
    src = edge_index[0]
    dst = edge_index[1]
    pad = E_PAD - N_EDGES
    ar = jnp.arange(pad, dtype=jnp.int32)
    src_p = jnp.concatenate([src, N_NODES + ar % (ROWS_PAD - N_NODES)])
    dst_p = jnp.concatenate([dst, ar % 8192])
    pk = ((src_p << 16) | dst_p).reshape(NW, NB, K)
    zeros = jnp.zeros((ROWS_PAD, DIM), jnp.float32)

    msg = _msg_matmul(X, W_msg.T, b_msg)
    p0, p1 = _sc_scatter(msg, pk, zeros)
    s = jnp.reshape(1.0 + eps, (1, 1)).astype(jnp.float32)
    return _final(s, X, p0, p1, W_lin.T, b_lin)


# speedup vs baseline: 1.1832x; 1.1832x over previous
import functools

import jax
import jax.numpy as jnp
from jax import lax
from jax.experimental import pallas as pl
from jax.experimental.pallas import tpu as pltpu
from jax.experimental.pallas import tpu_sc as plsc

N_NODES = 10000
DIM = 128
N_EDGES = 320000

K = 128
NC, NS = 2, 16
NW = NC * NS
NB = 80
NBLK = NB * NW
E_PAD = NBLK * K
ROWS_PAD = 10112
ZB = ROWS_PAD // NS
MMB = 2000


def _mm_kernel(x_ref, wt_ref, b_ref, o_ref):
    o_ref[...] = (
        jnp.dot(x_ref[...], wt_ref[...], preferred_element_type=jnp.float32)
        + b_ref[...]
    )


def _msg_matmul(x, wt, b):
    return pl.pallas_call(
        _mm_kernel,
        grid=(N_NODES // MMB,),
        in_specs=[
            pl.BlockSpec((MMB, DIM), lambda i: (i, 0)),
            pl.BlockSpec((DIM, DIM), lambda i: (0, 0)),
            pl.BlockSpec((1, DIM), lambda i: (0, 0)),
        ],
        out_specs=pl.BlockSpec((MMB, DIM), lambda i: (i, 0)),
        out_shape=jax.ShapeDtypeStruct((N_NODES, DIM), jnp.float32),
    )(x, wt, b.reshape(1, DIM))


def _final_kernel(s_ref, x_ref, p0_ref, p1_ref, wt_ref, b_ref, o_ref):
    t = x_ref[...] * s_ref[0, 0] + p0_ref[...] + p1_ref[...]
    y = (
        jnp.dot(t, wt_ref[...], preferred_element_type=jnp.float32)
        + b_ref[...]
    )
    o_ref[...] = jnp.maximum(y, 0.0)


def _final(s, x, p0, p1, wt, b):
    return pl.pallas_call(
        _final_kernel,
        grid=(N_NODES // MMB,),
        in_specs=[
            pl.BlockSpec(memory_space=pltpu.SMEM),
            pl.BlockSpec((MMB, DIM), lambda i: (i, 0)),
            pl.BlockSpec((MMB, DIM), lambda i: (i, 0)),
            pl.BlockSpec((MMB, DIM), lambda i: (i, 0)),
            pl.BlockSpec((DIM, DIM), lambda i: (0, 0)),
            pl.BlockSpec((1, DIM), lambda i: (0, 0)),
        ],
        out_specs=pl.BlockSpec((MMB, DIM), lambda i: (i, 0)),
        out_shape=jax.ShapeDtypeStruct((N_NODES, DIM), jnp.float32),
    )(s, x, p0, p1, wt, b.reshape(1, DIM))


def _sc_scatter(msg, pk, zeros):
    mesh = plsc.VectorSubcoreMesh(core_axis_name="c", subcore_axis_name="s")

    NSPLIT = 2
    NBUF = 2 * NSPLIT
    H = K // NSPLIT

    @functools.partial(
        pl.kernel,
        out_type=(jax.ShapeDtypeStruct((ROWS_PAD, DIM), jnp.float32),
                  jax.ShapeDtypeStruct((ROWS_PAD, DIM), jnp.float32)),
        mesh=mesh,
        scratch_types=(
            [pltpu.VMEM_SHARED((ROWS_PAD, DIM), jnp.float32),
             pltpu.VMEM((NB, K), jnp.int32)]
            + [pltpu.VMEM((H,), jnp.int32) for _ in range(2 * NBUF)]
            + [pltpu.VMEM((H, DIM), jnp.float32) for _ in range(NBUF)]
            + [pltpu.SemaphoreType.DMA for _ in range(NBUF + 1)]
        ),
    )
    def k(msg_hbm, pk_hbm, z_hbm, out0_hbm, out1_hbm, acc, pk_all, *rest):
        ds = rest[0:NBUF]
        ss = rest[NBUF:2 * NBUF]
        rr = rest[2 * NBUF:3 * NBUF]
        gg = rest[3 * NBUF:4 * NBUF]
        zs = rest[4 * NBUF]
        cid = lax.axis_index("c")
        sid = lax.axis_index("s")

        def unpack(row, q, dbuf, sbuf):
            for c in range(0, H, 16):
                v = pk_all[row, pl.ds(q * H + c, 16)]
                dbuf[pl.ds(c, 16)] = v & 0xFFFF
                sbuf[pl.ds(c, 16)] = v >> 16

        wid = cid * NS + sid

        pltpu.async_copy(z_hbm.at[pl.ds(sid * ZB, ZB)],
                         acc.at[pl.ds(sid * ZB, ZB)], zs)
        pltpu.sync_copy(pk_hbm.at[wid], pk_all)
        for b in range(NBUF):
            unpack(b // NSPLIT, b % NSPLIT, ds[b], ss[b])
            pltpu.async_copy(msg_hbm.at[ds[b]], rr[b], gg[b])
        pltpu.make_async_copy(z_hbm.at[pl.ds(sid * ZB, ZB)],
                              acc.at[pl.ds(sid * ZB, ZB)], zs).wait()
        plsc.subcore_barrier()

        @pl.loop(0, NB, step=2)
        def _(j):
            for b in range(NBUF):
                pltpu.make_async_copy(msg_hbm.at[ds[b]], rr[b], gg[b]).wait()
                pltpu.sync_copy(rr[b], acc.at[ss[b]], add=True)

                @pl.when(j + 2 + b // NSPLIT < NB)
                def _(b=b):
                    unpack(j + 2 + b // NSPLIT, b % NSPLIT, ds[b], ss[b])
                    pltpu.async_copy(msg_hbm.at[ds[b]], rr[b], gg[b])

        plsc.subcore_barrier()

        @pl.when(cid == 0)
        def _():
            pltpu.sync_copy(acc.at[pl.ds(sid * ZB, ZB)],
                            out0_hbm.at[pl.ds(sid * ZB, ZB)])

        @pl.when(cid == 1)
        def _():
            pltpu.sync_copy(acc.at[pl.ds(sid * ZB, ZB)],
                            out1_hbm.at[pl.ds(sid * ZB, ZB)])

    return k(msg, pk, zeros)


def kernel(X, edge_index, eps, W_msg, b_msg, W_lin, b_lin):
    src = edge_index[0]
    dst = edge_index[1]
    pad = E_PAD - N_EDGES
    ar = jnp.arange(pad, dtype=jnp.int32)
    src_p = jnp.concatenate([src, N_NODES + ar % (ROWS_PAD - N_NODES)])
    dst_p = jnp.concatenate([dst, ar % 8192])
    pk = ((src_p << 16) | dst_p).reshape(NW, NB, K)
    zeros = jnp.zeros((ROWS_PAD, DIM), jnp.float32)

    msg = _msg_matmul(X, W_msg.T, b_msg)
    p0, p1 = _sc_scatter(msg, pk, zeros)
    s = jnp.reshape(1.0 + eps, (1, 1)).astype(jnp.float32)
    return _final(s, X, p0, p1, W_lin.T, b_lin)


# speedup vs baseline: 1.2104x; 1.0230x over previous
import functools

import jax
import jax.numpy as jnp
from jax import lax
from jax.experimental import pallas as pl
from jax.experimental.pallas import tpu as pltpu
from jax.experimental.pallas import tpu_sc as plsc

N_NODES = 10000
DIM = 128
N_EDGES = 320000

K = 128
NC, NS = 2, 16
NW = NC * NS
NB = 80
NBLK = NB * NW
E_PAD = NBLK * K
ROWS_PAD = 10112
ZB = ROWS_PAD // NS
MMB = 5000


def _mm_kernel(x_ref, wt_ref, b_ref, o_ref):
    o_ref[...] = (
        jnp.dot(x_ref[...], wt_ref[...], preferred_element_type=jnp.float32)
        + b_ref[...]
    )


def _msg_matmul(x, wt, b):
    return pl.pallas_call(
        _mm_kernel,
        grid=(N_NODES // MMB,),
        in_specs=[
            pl.BlockSpec((MMB, DIM), lambda i: (i, 0)),
            pl.BlockSpec((DIM, DIM), lambda i: (0, 0)),
            pl.BlockSpec((1, DIM), lambda i: (0, 0)),
        ],
        out_specs=pl.BlockSpec((MMB, DIM), lambda i: (i, 0)),
        out_shape=jax.ShapeDtypeStruct((N_NODES, DIM), jnp.float32),
    )(x, wt, b.reshape(1, DIM))


def _final_kernel(s_ref, x_ref, p0_ref, p1_ref, wt_ref, b_ref, o_ref):
    t = x_ref[...] * s_ref[0, 0] + p0_ref[...] + p1_ref[...]
    y = (
        jnp.dot(t, wt_ref[...], preferred_element_type=jnp.float32)
        + b_ref[...]
    )
    o_ref[...] = jnp.maximum(y, 0.0)


def _final(s, x, p0, p1, wt, b):
    return pl.pallas_call(
        _final_kernel,
        grid=(N_NODES // MMB,),
        in_specs=[
            pl.BlockSpec(memory_space=pltpu.SMEM),
            pl.BlockSpec((MMB, DIM), lambda i: (i, 0)),
            pl.BlockSpec((MMB, DIM), lambda i: (i, 0)),
            pl.BlockSpec((MMB, DIM), lambda i: (i, 0)),
            pl.BlockSpec((DIM, DIM), lambda i: (0, 0)),
            pl.BlockSpec((1, DIM), lambda i: (0, 0)),
        ],
        out_specs=pl.BlockSpec((MMB, DIM), lambda i: (i, 0)),
        out_shape=jax.ShapeDtypeStruct((N_NODES, DIM), jnp.float32),
    )(s, x, p0, p1, wt, b.reshape(1, DIM))


def _sc_scatter(msg, pk, zeros):
    mesh = plsc.VectorSubcoreMesh(core_axis_name="c", subcore_axis_name="s")

    NSPLIT = 2
    NBUF = 2 * NSPLIT
    H = K // NSPLIT

    @functools.partial(
        pl.kernel,
        out_type=(jax.ShapeDtypeStruct((ROWS_PAD, DIM), jnp.float32),
                  jax.ShapeDtypeStruct((ROWS_PAD, DIM), jnp.float32)),
        mesh=mesh,
        scratch_types=(
            [pltpu.VMEM_SHARED((ROWS_PAD, DIM), jnp.float32),
             pltpu.VMEM((NB, K), jnp.int32)]
            + [pltpu.VMEM((H,), jnp.int32) for _ in range(2 * NBUF)]
            + [pltpu.VMEM((H, DIM), jnp.float32) for _ in range(NBUF)]
            + [pltpu.SemaphoreType.DMA for _ in range(NBUF + 1)]
        ),
    )
    def k(msg_hbm, pk_hbm, z_hbm, out0_hbm, out1_hbm, acc, pk_all, *rest):
        ds = rest[0:NBUF]
        ss = rest[NBUF:2 * NBUF]
        rr = rest[2 * NBUF:3 * NBUF]
        gg = rest[3 * NBUF:4 * NBUF]
        zs = rest[4 * NBUF]
        cid = lax.axis_index("c")
        sid = lax.axis_index("s")

        def unpack(row, q, dbuf, sbuf):
            for c in range(0, H, 16):
                v = pk_all[row, pl.ds(q * H + c, 16)]
                dbuf[pl.ds(c, 16)] = v & 0xFFFF
                sbuf[pl.ds(c, 16)] = v >> 16

        wid = cid * NS + sid

        pltpu.async_copy(z_hbm.at[pl.ds(sid * ZB, ZB)],
                         acc.at[pl.ds(sid * ZB, ZB)], zs)
        pltpu.sync_copy(pk_hbm.at[wid], pk_all)
        for b in range(NBUF):
            unpack(b // NSPLIT, b % NSPLIT, ds[b], ss[b])
            pltpu.async_copy(msg_hbm.at[ds[b]], rr[b], gg[b])
        pltpu.make_async_copy(z_hbm.at[pl.ds(sid * ZB, ZB)],
                              acc.at[pl.ds(sid * ZB, ZB)], zs).wait()
        plsc.subcore_barrier()

        @pl.loop(0, NB, step=2)
        def _(j):
            for b in range(NBUF):
                pltpu.make_async_copy(msg_hbm.at[ds[b]], rr[b], gg[b]).wait()
                pltpu.sync_copy(rr[b], acc.at[ss[b]], add=True)

                @pl.when(j + 2 + b // NSPLIT < NB)
                def _(b=b):
                    unpack(j + 2 + b // NSPLIT, b % NSPLIT, ds[b], ss[b])
                    pltpu.async_copy(msg_hbm.at[ds[b]], rr[b], gg[b])

        plsc.subcore_barrier()

        @pl.when(cid == 0)
        def _():
            pltpu.sync_copy(acc.at[pl.ds(sid * ZB, ZB)],
                            out0_hbm.at[pl.ds(sid * ZB, ZB)])

        @pl.when(cid == 1)
        def _():
            pltpu.sync_copy(acc.at[pl.ds(sid * ZB, ZB)],
                            out1_hbm.at[pl.ds(sid * ZB, ZB)])

    return k(msg, pk, zeros)


def kernel(X, edge_index, eps, W_msg, b_msg, W_lin, b_lin):
    src = edge_index[0]
    dst = edge_index[1]
    pad = E_PAD - N_EDGES
    ar = jnp.arange(pad, dtype=jnp.int32)
    src_p = jnp.concatenate([src, N_NODES + ar % (ROWS_PAD - N_NODES)])
    dst_p = jnp.concatenate([dst, ar % 8192])
    pk = ((src_p << 16) | dst_p).reshape(NW, NB, K)
    zeros = jnp.zeros((ROWS_PAD, DIM), jnp.float32)

    msg = _msg_matmul(X, W_msg.T, b_msg)
    p0, p1 = _sc_scatter(msg, pk, zeros)
    s = jnp.reshape(1.0 + eps, (1, 1)).astype(jnp.float32)
    return _final(s, X, p0, p1, W_lin.T, b_lin)
